# baseline (speedup 1.0000x reference)
import jax
import jax.numpy as jnp
from jax import lax
from jax.experimental import pallas as pl
from jax.experimental.pallas import tpu as pltpu
from jax.experimental.pallas import tpu_sc as plsc

N_ROWS = 100000
DIM = 128
NW = 32
W = 3200
CH = 64
NCH = W // CH
NBUF = 10
STEPS = NCH // NBUF


def _emb_body(idx_hbm, table_hbm, out_hbm, idx_v, table_sh, bufs, gsem, wsem, lsem):
    wid = lax.axis_index("s") * 2 + lax.axis_index("c")
    base = jnp.minimum(wid * W, N_ROWS - W)

    cp_i = pltpu.async_copy(idx_hbm.at[pl.ds(base, W)], idx_v, lsem)

    @pl.when(lax.axis_index("s") == 0)
    def _():
        pltpu.sync_copy(table_hbm, table_sh)

    cp_i.wait()
    plsc.subcore_barrier()

    def gather(c, b):
        return pltpu.make_async_copy(
            table_sh.at[idx_v.at[pl.ds(c * CH, CH)]],
            bufs.at[pl.ds(b * CH, CH)],
            gsem.at[b],
        )

    def write(c, b):
        return pltpu.make_async_copy(
            bufs.at[pl.ds(b * CH, CH)],
            out_hbm.at[pl.ds(base + c * CH, CH)],
            wsem.at[b],
        )

    for b in range(NBUF):
        gather(b, b).start()

    def step(s, carry):
        for b in range(NBUF):
            c = s * NBUF + b
            gather(c, b).wait()
            write(c, b).start()
        for b in range(NBUF):
            c = s * NBUF + b
            write(c, b).wait()

            @pl.when(c + NBUF < NCH)
            def _():
                gather(c + NBUF, b).start()

        return carry

    lax.fori_loop(0, STEPS, step, 0)


@jax.jit
def kernel(node_type, table):
    mesh = plsc.VectorSubcoreMesh(core_axis_name="c", subcore_axis_name="s")
    k = pl.kernel(
        _emb_body,
        out_type=jax.ShapeDtypeStruct((N_ROWS, DIM), jnp.float32),
        mesh=mesh,
        scratch_types=[
            pltpu.VMEM((W,), jnp.int32),
            pltpu.VMEM_SHARED((100, DIM), jnp.float32),
            pltpu.VMEM((NBUF * CH, DIM), jnp.float32),
            pltpu.SemaphoreType.DMA((NBUF,)),
            pltpu.SemaphoreType.DMA((NBUF,)),
            pltpu.SemaphoreType.DMA,
        ],
    )
    return k(node_type.astype(jnp.int32), table)


# speedup vs baseline: 1.0022x; 1.0022x over previous
import jax
import jax.numpy as jnp
from jax import lax
from jax.experimental import pallas as pl
from jax.experimental.pallas import tpu as pltpu
from jax.experimental.pallas import tpu_sc as plsc

N_ROWS = 100000
DIM = 128
NW = 32
W = 3200
CH = 80
NCH = W // CH
NBUF = 8
STEPS = NCH // NBUF


def _emb_body(idx_hbm, table_hbm, out_hbm, idx_v, table_sh, bufs, gsem, wsem, lsem):
    wid = lax.axis_index("s") * 2 + lax.axis_index("c")
    base = jnp.minimum(wid * W, N_ROWS - W)

    cp_i = pltpu.async_copy(idx_hbm.at[pl.ds(base, W)], idx_v, lsem)

    @pl.when(lax.axis_index("s") == 0)
    def _():
        pltpu.sync_copy(table_hbm, table_sh)

    cp_i.wait()
    plsc.subcore_barrier()

    def gather(c, b):
        return pltpu.make_async_copy(
            table_sh.at[idx_v.at[pl.ds(c * CH, CH)]],
            bufs.at[pl.ds(b * CH, CH)],
            gsem.at[b],
        )

    def write(c, b):
        return pltpu.make_async_copy(
            bufs.at[pl.ds(b * CH, CH)],
            out_hbm.at[pl.ds(base + c * CH, CH)],
            wsem.at[b],
        )

    for b in range(NBUF):
        gather(b, b).start()

    def step(s, carry):
        for b in range(NBUF):
            c = s * NBUF + b
            gather(c, b).wait()
            write(c, b).start()
        for b in range(NBUF):
            c = s * NBUF + b
            write(c, b).wait()

            @pl.when(c + NBUF < NCH)
            def _():
                gather(c + NBUF, b).start()

        return carry

    lax.fori_loop(0, STEPS, step, 0)


@jax.jit
def kernel(node_type, table):
    mesh = plsc.VectorSubcoreMesh(core_axis_name="c", subcore_axis_name="s")
    k = pl.kernel(
        _emb_body,
        out_type=jax.ShapeDtypeStruct((N_ROWS, DIM), jnp.float32),
        mesh=mesh,
        scratch_types=[
            pltpu.VMEM((W,), jnp.int32),
            pltpu.VMEM_SHARED((100, DIM), jnp.float32),
            pltpu.VMEM((NBUF * CH, DIM), jnp.float32),
            pltpu.SemaphoreType.DMA((NBUF,)),
            pltpu.SemaphoreType.DMA((NBUF,)),
            pltpu.SemaphoreType.DMA,
        ],
    )
    return k(node_type.astype(jnp.int32), table)
